# Initial kernel scaffold; baseline (speedup 1.0000x reference)
import jax
import jax.numpy as jnp
from jax.experimental import pallas as pl


def kernel(m2g_efeat, grid_nfeat, mesh_nfeat, edge_index, e_W1, e_b1, e_W2, e_b2, e_gamma, e_beta, n_W1, n_b1, n_W2, n_b2, n_gamma, n_beta):
    raise NotImplementedError("write your pallas kernel here")


# speedup vs baseline: 3.4440x; 3.4440x over previous
import functools

import jax
import jax.numpy as jnp
from jax import lax
from jax.experimental import pallas as pl
from jax.experimental.pallas import tpu as pltpu
from jax.experimental.pallas import tpu_sc as plsc

_CH = 128
_EPS = 1e-5


def _sc_gather(mesh_nfeat, grid_nfeat, src, dst):
    E = src.shape[0]
    D = mesh_nfeat.shape[1]
    info = plsc.get_sparse_core_info()
    NC, NS = info.num_cores, info.num_subcores
    NW = NC * NS
    n_chunks = E // _CH
    max_t = (n_chunks + NW - 1) // NW
    mesh = plsc.VectorSubcoreMesh(core_axis_name="c", subcore_axis_name="s")

    @functools.partial(
        pl.kernel,
        out_type=(jax.ShapeDtypeStruct((E, D), jnp.float32),
                  jax.ShapeDtypeStruct((E, D), jnp.float32)),
        mesh=mesh,
        scratch_types=[
            pltpu.VMEM((_CH,), jnp.int32),
            pltpu.VMEM((_CH,), jnp.int32),
            pltpu.VMEM((_CH, D), jnp.float32),
            pltpu.VMEM((_CH, D), jnp.float32),
            pltpu.SemaphoreType.DMA,
            pltpu.SemaphoreType.DMA,
        ],
    )
    def k(mesh_hbm, grid_hbm, src_hbm, dst_hbm, ms_out, gd_out,
          sidx, didx, srows, drows, sem0, sem1):
        wid = lax.axis_index("s") * NC + lax.axis_index("c")

        @pl.loop(0, max_t)
        def _(t):
            c = wid + NW * t

            @pl.when(c < n_chunks)
            def _():
                off = c * _CH
                pltpu.sync_copy(src_hbm.at[pl.ds(off, _CH)], sidx)
                pltpu.sync_copy(dst_hbm.at[pl.ds(off, _CH)], didx)
                cs = pltpu.async_copy(mesh_hbm.at[sidx], srows, sem0)
                cd = pltpu.async_copy(grid_hbm.at[didx], drows, sem1)
                cs.wait()
                pltpu.sync_copy(srows, ms_out.at[pl.ds(off, _CH)])
                cd.wait()
                pltpu.sync_copy(drows, gd_out.at[pl.ds(off, _CH)])

    return k(mesh_nfeat, grid_nfeat, src, dst)


def _sc_scatter(efeat, dst, n_out):
    E, D = efeat.shape
    info = plsc.get_sparse_core_info()
    NC, NS = info.num_cores, info.num_subcores
    NW = NC * NS
    n_chunks = E // _CH
    max_t = (n_chunks + NW - 1) // NW
    n_full = n_out // _CH
    n_tail = n_out - n_full * _CH
    max_r = (n_full + NS - 1) // NS
    mesh = plsc.VectorSubcoreMesh(core_axis_name="c", subcore_axis_name="s")

    @functools.partial(
        pl.kernel,
        out_type=jax.ShapeDtypeStruct((NC, n_out, D), jnp.float32),
        mesh=mesh,
        scratch_types=[
            pltpu.VMEM((_CH,), jnp.int32),
            pltpu.VMEM((_CH, D), jnp.float32),
            pltpu.VMEM_SHARED((n_out, D), jnp.float32),
            pltpu.SemaphoreType.DMA,
        ],
    )
    def k(efeat_hbm, dst_hbm, out_hbm, didx, rows, agg_sh, sem):
        cid = lax.axis_index("c")
        sid = lax.axis_index("s")
        wid = sid * NC + cid

        @pl.loop(0, _CH)
        def _(i):
            for j in range(D // 16):
                rows[i, pl.ds(j * 16, 16)] = jnp.zeros((16,), jnp.float32)

        @pl.loop(0, max_r)
        def _(t):
            r = sid + NS * t

            @pl.when(r < n_full)
            def _():
                pltpu.sync_copy(rows, agg_sh.at[pl.ds(r * _CH, _CH)])

        if n_tail:
            @pl.when(sid == NS - 1)
            def _():
                pltpu.sync_copy(rows.at[pl.ds(0, n_tail)],
                                agg_sh.at[pl.ds(n_full * _CH, n_tail)])

        plsc.subcore_barrier()

        @pl.loop(0, max_t)
        def _(t):
            c = wid + NW * t

            @pl.when(c < n_chunks)
            def _():
                off = c * _CH
                pltpu.sync_copy(dst_hbm.at[pl.ds(off, _CH)], didx)
                pltpu.sync_copy(efeat_hbm.at[pl.ds(off, _CH)], rows)
                pltpu.sync_copy(rows, agg_sh.at[didx], add=True)

        plsc.subcore_barrier()

        @pl.loop(0, max_r)
        def _(t):
            r = sid + NS * t

            @pl.when(r < n_full)
            def _():
                pltpu.sync_copy(agg_sh.at[pl.ds(r * _CH, _CH)],
                                out_hbm.at[cid, pl.ds(r * _CH, _CH)])

        if n_tail:
            @pl.when(sid == NS - 1)
            def _():
                pltpu.sync_copy(agg_sh.at[pl.ds(n_full * _CH, n_tail)],
                                out_hbm.at[cid, pl.ds(n_full * _CH, n_tail)])

    return k(efeat, dst)


def _edge_mlp_body(m2g, ms, gd, w1a, w1b, w1c, b1, w2, b2, g, b, out):
    x = (jnp.dot(m2g[...], w1a[...], preferred_element_type=jnp.float32)
         + jnp.dot(ms[...], w1b[...], preferred_element_type=jnp.float32)
         + jnp.dot(gd[...], w1c[...], preferred_element_type=jnp.float32)
         + b1[...])
    h = x * jax.nn.sigmoid(x)
    e = jnp.dot(h, w2[...], preferred_element_type=jnp.float32) + b2[...]
    mu = jnp.mean(e, axis=-1, keepdims=True)
    var = jnp.mean(jnp.square(e - mu), axis=-1, keepdims=True)
    out[...] = (e - mu) * lax.rsqrt(var + _EPS) * g[...] + b[...]


def _edge_mlp(m2g, msrc, gdst, w1a, w1b, w1c, b1, w2, b2, g, b, tile):
    E, D = m2g.shape
    H = w1a.shape[1]
    grid = (E // tile,)
    row = lambda i: (i, 0)
    fix = lambda i: (0, 0)
    return pl.pallas_call(
        _edge_mlp_body,
        grid=grid,
        in_specs=[
            pl.BlockSpec((tile, D), row),
            pl.BlockSpec((tile, D), row),
            pl.BlockSpec((tile, D), row),
            pl.BlockSpec((D, H), fix),
            pl.BlockSpec((D, H), fix),
            pl.BlockSpec((D, H), fix),
            pl.BlockSpec((1, H), fix),
            pl.BlockSpec((H, D), fix),
            pl.BlockSpec((1, D), fix),
            pl.BlockSpec((1, D), fix),
            pl.BlockSpec((1, D), fix),
        ],
        out_specs=pl.BlockSpec((tile, D), row),
        out_shape=jax.ShapeDtypeStruct((E, D), jnp.float32),
    )(m2g, msrc, gdst, w1a, w1b, w1c, b1, w2, b2, g, b)


def _node_mlp_body(aggp, grid, w1a, w1b, b1, w2, b2, g, b, out):
    a = aggp[...]
    agg = a[0] + a[1]
    gn = grid[...]
    x = (jnp.dot(agg, w1a[...], preferred_element_type=jnp.float32)
         + jnp.dot(gn, w1b[...], preferred_element_type=jnp.float32)
         + b1[...])
    h = x * jax.nn.sigmoid(x)
    e = jnp.dot(h, w2[...], preferred_element_type=jnp.float32) + b2[...]
    mu = jnp.mean(e, axis=-1, keepdims=True)
    var = jnp.mean(jnp.square(e - mu), axis=-1, keepdims=True)
    out[...] = (e - mu) * lax.rsqrt(var + _EPS) * g[...] + b[...] + gn


def _node_mlp(aggp, grid_nfeat, w1a, w1b, b1, w2, b2, g, b, tile):
    NC, N, D = aggp.shape
    H = w1a.shape[1]
    grid = (N // tile,)
    fix = lambda i: (0, 0)
    return pl.pallas_call(
        _node_mlp_body,
        grid=grid,
        in_specs=[
            pl.BlockSpec((NC, tile, D), lambda i: (0, i, 0)),
            pl.BlockSpec((tile, D), lambda i: (i, 0)),
            pl.BlockSpec((D, H), fix),
            pl.BlockSpec((D, H), fix),
            pl.BlockSpec((1, H), fix),
            pl.BlockSpec((H, D), fix),
            pl.BlockSpec((1, D), fix),
            pl.BlockSpec((1, D), fix),
            pl.BlockSpec((1, D), fix),
        ],
        out_specs=pl.BlockSpec((tile, D), lambda i: (i, 0)),
        out_shape=jax.ShapeDtypeStruct((N, D), jnp.float32),
    )(aggp, grid_nfeat, w1a, w1b, b1, w2, b2, g, b)


def kernel(m2g_efeat, grid_nfeat, mesh_nfeat, edge_index,
           e_W1, e_b1, e_W2, e_b2, e_gamma, e_beta,
           n_W1, n_b1, n_W2, n_b2, n_gamma, n_beta):
    D = m2g_efeat.shape[1]
    n_dst = grid_nfeat.shape[0]
    src = edge_index[0].astype(jnp.int32)
    dst = edge_index[1].astype(jnp.int32)

    msrc, gdst = _sc_gather(mesh_nfeat, grid_nfeat, src, dst)

    efeat = _edge_mlp(
        m2g_efeat, msrc, gdst,
        e_W1[:D], e_W1[D:2 * D], e_W1[2 * D:],
        e_b1.reshape(1, -1), e_W2, e_b2.reshape(1, -1),
        e_gamma.reshape(1, -1), e_beta.reshape(1, -1), tile=2000)

    aggp = _sc_scatter(efeat, dst, n_dst)

    return _node_mlp(
        aggp, grid_nfeat,
        n_W1[:D], n_W1[D:],
        n_b1.reshape(1, -1), n_W2, n_b2.reshape(1, -1),
        n_gamma.reshape(1, -1), n_beta.reshape(1, -1), tile=1000)
